# baseline (speedup 1.0000x reference)
import functools

import jax
import jax.numpy as jnp
from jax import lax
from jax.experimental import pallas as pl
from jax.experimental.pallas import tpu as pltpu
from jax.experimental.pallas import tpu_sc as plsc

_G = 128


def _make_gather(B, V, D, NC, NS):
    NW = NC * NS
    b_per_w = B // NW
    CHUNK = 1024
    n_chunks = b_per_w // CHUNK
    n_g = CHUNK // _G
    assert b_per_w % CHUNK == 0

    mesh = plsc.VectorSubcoreMesh(core_axis_name="c", subcore_axis_name="s")

    @functools.partial(
        pl.kernel,
        mesh=mesh,
        out_type=jax.ShapeDtypeStruct((B, D), jnp.float32),
        scratch_types=[
            pltpu.VMEM((n_g, _G), jnp.int32),
            pltpu.VMEM((CHUNK, D), jnp.float32),
            pltpu.SemaphoreType.DMA,
        ],
        compiler_params=pltpu.CompilerParams(use_tc_tiling_on_sc=False),
    )
    def gather_kernel(idx_hbm, table_hbm, out_hbm, idx_v, rows_v, sem):
        wid = lax.axis_index("s") * NC + lax.axis_index("c")
        base_row = wid * (b_per_w // _G)
        base_out = wid * b_per_w

        def body(i, carry):
            pltpu.sync_copy(idx_hbm.at[pl.ds(base_row + i * n_g, n_g)], idx_v)
            copies = []
            for j in range(n_g):
                copies.append(pltpu.async_copy(
                    table_hbm.at[idx_v.at[j]],
                    rows_v.at[pl.ds(j * _G, _G)],
                    sem,
                ))
            for c in copies:
                c.wait()
            pltpu.sync_copy(rows_v, out_hbm.at[pl.ds(base_out + i * CHUNK, CHUNK)])
            return carry

        lax.fori_loop(0, n_chunks, body, 0)

    return gather_kernel


def kernel(x, table):
    Bb, S = x.shape
    V, D = table.shape
    B = Bb * S
    info = plsc.get_sparse_core_info()
    gather = _make_gather(B, V, D, info.num_cores, info.num_subcores)
    idx = x.reshape(B // _G, _G)
    out = gather(idx, table)
    return out.reshape(Bb, S, D)


# speedup vs baseline: 1.5686x; 1.5686x over previous
import functools

import jax
import jax.numpy as jnp
from jax import lax
from jax.experimental import pallas as pl
from jax.experimental.pallas import tpu as pltpu
from jax.experimental.pallas import tpu_sc as plsc

_LANES = 128


def _make_gather(B, V, D, NC, NS):
    NW = NC * NS
    b_per_w = B // NW
    CHUNK = 512
    n_steps = b_per_w // CHUNK
    G = 128
    n_g = CHUNK // G
    assert b_per_w % CHUNK == 0 and CHUNK % G == 0

    mesh = plsc.VectorSubcoreMesh(core_axis_name="c", subcore_axis_name="s")

    @functools.partial(
        pl.kernel,
        mesh=mesh,
        out_type=jax.ShapeDtypeStruct((B, _LANES), jnp.float32),
        scratch_types=[
            pltpu.VMEM((CHUNK,), jnp.int32),
            pltpu.VMEM((CHUNK, _LANES), jnp.float32),
            pltpu.SemaphoreType.DMA,
        ],
    )
    def gather_kernel(idx_hbm, tab_hbm, out_hbm, idx_v, rows_v, sem):
        wid = lax.axis_index("s") * NC + lax.axis_index("c")
        base = wid * b_per_w

        def body(i, carry):
            off = base + i * CHUNK
            pltpu.sync_copy(idx_hbm.at[pl.ds(off, CHUNK)], idx_v)
            copies = []
            for j in range(n_g):
                copies.append(pltpu.async_copy(
                    tab_hbm.at[idx_v.at[pl.ds(j * G, G)]],
                    rows_v.at[pl.ds(j * G, G)],
                    sem,
                ))
            for c in copies:
                c.wait()
            pltpu.sync_copy(rows_v, out_hbm.at[pl.ds(off, CHUNK)])
            return carry

        lax.fori_loop(0, n_steps, body, 0)

    return gather_kernel


def kernel(x, table):
    Bb, S = x.shape
    V, D = table.shape
    B = Bb * S
    info = plsc.get_sparse_core_info()
    gather = _make_gather(B, V, D, info.num_cores, info.num_subcores)
    tab_pad = jnp.pad(table, ((0, 0), (0, _LANES - D)))
    out = gather(x.reshape(B), tab_pad)
    return out[:, :D].reshape(Bb, S, D)


# speedup vs baseline: 1.6536x; 1.0542x over previous
import functools

import jax
import jax.numpy as jnp
from jax import lax
from jax.experimental import pallas as pl
from jax.experimental.pallas import tpu as pltpu
from jax.experimental.pallas import tpu_sc as plsc

_LANES = 128


def _make_gather(B, V, D, NC, NS):
    NW = NC * NS
    b_per_w = B // NW
    CHUNK = 256
    n_steps = b_per_w // CHUNK
    G = 128
    n_g = CHUNK // G
    NBUF = 2
    assert b_per_w % CHUNK == 0 and CHUNK % G == 0 and n_steps % NBUF == 0

    mesh = plsc.VectorSubcoreMesh(core_axis_name="c", subcore_axis_name="s")

    @functools.partial(
        pl.kernel,
        mesh=mesh,
        out_type=jax.ShapeDtypeStruct((B, _LANES), jnp.float32),
        scratch_types=[
            pltpu.VMEM((NBUF, CHUNK), jnp.int32),
            pltpu.VMEM((NBUF, CHUNK, _LANES), jnp.float32),
            [pltpu.SemaphoreType.DMA] * NBUF,
            [pltpu.SemaphoreType.DMA] * NBUF,
        ],
    )
    def gather_kernel(idx_hbm, tab_hbm, out_hbm, idx_v, rows_v, gsems, osems):
        wid = lax.axis_index("s") * NC + lax.axis_index("c")
        base = wid * b_per_w

        def fire(step, slot):
            off = base + step * CHUNK
            pltpu.sync_copy(idx_hbm.at[pl.ds(off, CHUNK)], idx_v.at[slot])
            for j in range(n_g):
                pltpu.async_copy(
                    tab_hbm.at[idx_v.at[slot, pl.ds(j * G, G)]],
                    rows_v.at[slot, pl.ds(j * G, G)],
                    gsems[slot],
                )

        def drain_and_write(step, slot):
            off = base + step * CHUNK
            for j in range(n_g):
                pltpu.make_async_copy(
                    tab_hbm.at[idx_v.at[slot, pl.ds(j * G, G)]],
                    rows_v.at[slot, pl.ds(j * G, G)],
                    gsems[slot],
                ).wait()
            pltpu.async_copy(rows_v.at[slot], out_hbm.at[pl.ds(off, CHUNK)],
                             osems[slot])

        def wait_write(step, slot):
            off = base + step * CHUNK
            pltpu.make_async_copy(rows_v.at[slot],
                                  out_hbm.at[pl.ds(off, CHUNK)],
                                  osems[slot]).wait()

        for slot in range(NBUF):
            fire(slot, slot)

        def body(i, carry):
            slot = lax.rem(i, NBUF)

            for s in range(NBUF):
                @pl.when(slot == s)
                def _():
                    drain_and_write(i, s)

            for s in range(NBUF):
                @pl.when(jnp.logical_and(slot == s, i + NBUF < n_steps))
                def _():
                    wait_write(i, s)
                    fire(i + NBUF, s)

            return carry

        lax.fori_loop(0, n_steps, body, 0)

        for slot in range(NBUF):
            step = n_steps - NBUF + slot
            wait_write(step, slot)

    return gather_kernel


def kernel(x, table):
    Bb, S = x.shape
    V, D = table.shape
    B = Bb * S
    info = plsc.get_sparse_core_info()
    gather = _make_gather(B, V, D, info.num_cores, info.num_subcores)
    tab_pad = jnp.pad(table, ((0, 0), (0, _LANES - D)))
    out = gather(x.reshape(B), tab_pad)
    return out[:, :D].reshape(Bb, S, D)


# speedup vs baseline: 1.6593x; 1.0034x over previous
import functools

import jax
import jax.numpy as jnp
from jax import lax
from jax.experimental import pallas as pl
from jax.experimental.pallas import tpu as pltpu
from jax.experimental.pallas import tpu_sc as plsc

_LANES = 128


def _make_gather(B, V, D, NC, NS):
    NW = NC * NS
    b_per_w = B // NW
    CHUNK = 128
    n_steps = b_per_w // CHUNK
    G = 128
    n_g = CHUNK // G
    NBUF = 4
    assert b_per_w % CHUNK == 0 and CHUNK % G == 0 and n_steps > NBUF

    mesh = plsc.VectorSubcoreMesh(core_axis_name="c", subcore_axis_name="s")

    @functools.partial(
        pl.kernel,
        mesh=mesh,
        out_type=jax.ShapeDtypeStruct((B, _LANES), jnp.float32),
        scratch_types=[
            pltpu.VMEM((NBUF, CHUNK), jnp.int32),
            pltpu.VMEM((NBUF, CHUNK, _LANES), jnp.float32),
            [pltpu.SemaphoreType.DMA] * NBUF,
            [pltpu.SemaphoreType.DMA] * NBUF,
        ],
    )
    def gather_kernel(idx_hbm, tab_hbm, out_hbm, idx_v, rows_v, gsems, osems):
        wid = lax.axis_index("s") * NC + lax.axis_index("c")
        base = wid * b_per_w

        def fire(step, slot):
            off = base + step * CHUNK
            pltpu.sync_copy(idx_hbm.at[pl.ds(off, CHUNK)], idx_v.at[slot])
            for j in range(n_g):
                pltpu.async_copy(
                    tab_hbm.at[idx_v.at[slot, pl.ds(j * G, G)]],
                    rows_v.at[slot, pl.ds(j * G, G)],
                    gsems[slot],
                )

        def drain_gathers(slot):
            for j in range(n_g):
                pltpu.make_async_copy(
                    tab_hbm.at[idx_v.at[slot, pl.ds(j * G, G)]],
                    rows_v.at[slot, pl.ds(j * G, G)],
                    gsems[slot],
                ).wait()

        def fire_write(step, slot):
            off = base + step * CHUNK
            pltpu.async_copy(rows_v.at[slot], out_hbm.at[pl.ds(off, CHUNK)],
                             osems[slot])

        def wait_write(step, slot):
            off = base + step * CHUNK
            pltpu.make_async_copy(rows_v.at[slot],
                                  out_hbm.at[pl.ds(off, CHUNK)],
                                  osems[slot]).wait()

        for s in range(NBUF - 1):
            fire(s, s)

        def body(i, carry):
            s_cur = lax.rem(i, NBUF)
            s_pre = lax.rem(i + NBUF - 1, NBUF)
            for s in range(NBUF):
                @pl.when(s_cur == s)
                def _():
                    drain_gathers(s)
                    fire_write(i, s)

                @pl.when(jnp.logical_and(s_pre == s, i + NBUF - 1 < n_steps))
                def _():
                    @pl.when(i >= 1)
                    def _():
                        wait_write(i - 1, s)
                    fire(i + NBUF - 1, s)
            return carry

        lax.fori_loop(0, n_steps, body, 0)

        for k in range(NBUF):
            step = n_steps - NBUF + k
            wait_write(step, step % NBUF)

    return gather_kernel


def kernel(x, table):
    Bb, S = x.shape
    V, D = table.shape
    B = Bb * S
    info = plsc.get_sparse_core_info()
    gather = _make_gather(B, V, D, info.num_cores, info.num_subcores)
    tab_pad = jnp.pad(table, ((0, 0), (0, _LANES - D)))
    out = gather(x.reshape(B), tab_pad)
    return out[:, :D].reshape(Bb, S, D)
